# Initial kernel scaffold; baseline (speedup 1.0000x reference)
import jax
import jax.numpy as jnp
from jax.experimental import pallas as pl


def kernel(dists_same, dists_anti, dists_ne, senders_same, receivers_same, senders_anti, receivers_anti, senders_ne, receivers_ne, params):
    raise NotImplementedError("write your pallas kernel here")


# speedup vs baseline: 1.4434x; 1.4434x over previous
import functools
import numpy as np
import jax
import jax.numpy as jnp
from jax import lax
from jax.experimental import pallas as pl
from jax.experimental.pallas import tpu as pltpu

N_ELEC = 10000
N_NUC = 16
EMB = 128
KER = 64
DFD = 32
CUTOFF = 10.0
N_LAYERS = 3
LOG2 = float(np.log(2.0))

_delta = 1.0 / (2 * DFD)
_qs = np.linspace(_delta, 1.0 - _delta, DFD)
_mus = CUTOFF * _qs ** 2
_sigmas = (1.0 + CUTOFF * _qs) / 7.0
_MUS = np.zeros((1, 128), np.float32)
_MUS[0, :DFD] = _mus
_ISIG2 = np.zeros((1, 128), np.float32)
_ISIG2[0, :DFD] = 1.0 / _sigmas ** 2
_BMASK = np.zeros((1, 128), np.float32)
_BMASK[0, :DFD] = 1.0

EDGE_B = 2000
NODE_B = 2000


def _ssp(x):
    return jnp.logaddexp(x, 0.0) - LOG2


def _pad2(w, r, c):
    return jnp.zeros((r, c), jnp.float32).at[: w.shape[0], : w.shape[1]].set(w)


def _pad_row(b, c):
    return jnp.zeros((1, c), jnp.float32).at[0, : b.shape[0]].set(b)


def _edge_body(d_ref, mus_ref, isig_ref, bmask_ref,
               w1_ref, b1_ref, w2_ref, b2_ref, w3_ref, out_ref):
    d = d_ref[:]
    env = d * d * jnp.exp(-d)
    t = d - mus_ref[:]
    feat = env * jnp.exp(-(t * t) * isig_ref[:]) * bmask_ref[:]
    h1 = _ssp(jnp.dot(feat, w1_ref[:], preferred_element_type=jnp.float32) + b1_ref[:])
    h2 = _ssp(jnp.dot(h1, w2_ref[:], preferred_element_type=jnp.float32) + b2_ref[:])
    out_ref[:] = jnp.dot(h2, w3_ref[:], preferred_element_type=jnp.float32)


def _edge_ne_body(d_ref, s_ref, mus_ref, isig_ref, bmask_ref,
                  w1_ref, b1_ref, w2_ref, b2_ref, w3_ref, y_ref, out_ref):
    d = d_ref[:]
    env = d * d * jnp.exp(-d)
    t = d - mus_ref[:]
    feat = env * jnp.exp(-(t * t) * isig_ref[:]) * bmask_ref[:]
    h1 = _ssp(jnp.dot(feat, w1_ref[:], preferred_element_type=jnp.float32) + b1_ref[:])
    h2 = _ssp(jnp.dot(h1, w2_ref[:], preferred_element_type=jnp.float32) + b2_ref[:])
    we = jnp.dot(h2, w3_ref[:], preferred_element_type=jnp.float32)
    s = s_ref[:]
    lanes = lax.broadcasted_iota(jnp.int32, (s.shape[0], 128), 1)
    onehot = (lanes == s).astype(jnp.float32)
    hx = jnp.dot(onehot, y_ref[:], preferred_element_type=jnp.float32)
    out_ref[:] = we * hx


def _node_h_body(e_ref, w1s_ref, b1s_ref, w2s_ref,
                 w1a_ref, b1a_ref, w2a_ref, hs_ref, ha_ref):
    e = e_ref[:]
    hs = _ssp(jnp.dot(e, w1s_ref[:], preferred_element_type=jnp.float32) + b1s_ref[:])
    hs_ref[:] = jnp.dot(hs, w2s_ref[:], preferred_element_type=jnp.float32)
    ha = _ssp(jnp.dot(e, w1a_ref[:], preferred_element_type=jnp.float32) + b1a_ref[:])
    ha_ref[:] = jnp.dot(ha, w2a_ref[:], preferred_element_type=jnp.float32)


def _update_body(e_ref, zs_ref, za_ref, zn_ref,
                 ws1_ref, bs1_ref, ws2_ref,
                 wa1_ref, ba1_ref, wa2_ref,
                 wn1_ref, bn1_ref, wn2_ref, out_ref):
    acc = e_ref[:]
    for z_ref, w1_ref, b1_ref, w2_ref in (
            (zs_ref, ws1_ref, bs1_ref, ws2_ref),
            (za_ref, wa1_ref, ba1_ref, wa2_ref),
            (zn_ref, wn1_ref, bn1_ref, wn2_ref)):
        h = _ssp(jnp.dot(z_ref[:], w1_ref[:], preferred_element_type=jnp.float32) + b1_ref[:])
        acc = acc + jnp.dot(h, w2_ref[:], preferred_element_type=jnp.float32)
    out_ref[:] = acc


def _full(shape):
    return pl.BlockSpec(shape, lambda i: tuple(0 for _ in shape))


def _edge_mlp(dists2d, wts, e_total):
    grid = (e_total // EDGE_B,)
    in_specs = [pl.BlockSpec((EDGE_B, 1), lambda i: (i, 0)),
                _full((1, 128)), _full((1, 128)), _full((1, 128)),
                _full((128, 128)), _full((1, 128)),
                _full((128, 128)), _full((1, 128)),
                _full((128, 64))]
    return pl.pallas_call(
        _edge_body, grid=grid, in_specs=in_specs,
        out_specs=pl.BlockSpec((EDGE_B, 64), lambda i: (i, 0)),
        out_shape=jax.ShapeDtypeStruct((e_total, 64), jnp.float32),
    )(dists2d, jnp.asarray(_MUS), jnp.asarray(_ISIG2), jnp.asarray(_BMASK), *wts)


def _edge_mlp_ne(dists2d, senders2d, wts, y128, e_total):
    grid = (e_total // EDGE_B,)
    in_specs = [pl.BlockSpec((EDGE_B, 1), lambda i: (i, 0)),
                pl.BlockSpec((EDGE_B, 1), lambda i: (i, 0)),
                _full((1, 128)), _full((1, 128)), _full((1, 128)),
                _full((128, 128)), _full((1, 128)),
                _full((128, 128)), _full((1, 128)),
                _full((128, 64)), _full((128, 64))]
    return pl.pallas_call(
        _edge_ne_body, grid=grid, in_specs=in_specs,
        out_specs=pl.BlockSpec((EDGE_B, 64), lambda i: (i, 0)),
        out_shape=jax.ShapeDtypeStruct((e_total, 64), jnp.float32),
    )(dists2d, senders2d, jnp.asarray(_MUS), jnp.asarray(_ISIG2),
      jnp.asarray(_BMASK), *wts, y128)


def _node_h(elec, wts):
    grid = (N_ELEC // NODE_B,)
    in_specs = [pl.BlockSpec((NODE_B, 128), lambda i: (i, 0)),
                _full((128, 128)), _full((1, 128)), _full((128, 64)),
                _full((128, 128)), _full((1, 128)), _full((128, 64))]
    out_specs = [pl.BlockSpec((NODE_B, 64), lambda i: (i, 0))] * 2
    return pl.pallas_call(
        _node_h_body, grid=grid, in_specs=in_specs, out_specs=out_specs,
        out_shape=[jax.ShapeDtypeStruct((N_ELEC, 64), jnp.float32)] * 2,
    )(elec, *wts)


def _update(elec, zs, za, zn, wts):
    grid = (N_ELEC // NODE_B,)
    in_specs = [pl.BlockSpec((NODE_B, 128), lambda i: (i, 0))] + \
               [pl.BlockSpec((NODE_B, 64), lambda i: (i, 0))] * 3 + \
               [_full((64, 128)), _full((1, 128)), _full((128, 128))] * 3
    return pl.pallas_call(
        _update_body, grid=grid, in_specs=in_specs,
        out_specs=pl.BlockSpec((NODE_B, 128), lambda i: (i, 0)),
        out_shape=jax.ShapeDtypeStruct((N_ELEC, EMB), jnp.float32),
    )(elec, zs, za, zn, *wts)


def _prep_w_mlp(layers, fold_row=None):
    w1 = _pad2(layers[0]['W'], 128, 128)
    b1 = _pad_row(layers[0]['b'], 128)
    w2 = _pad2(layers[1]['W'], 128, 128)
    b2 = _pad_row(layers[1]['b'], 128)
    w3 = layers[2]['W']
    if fold_row is not None:
        w3 = w3 * fold_row[None, :]
    w3 = _pad2(w3, 128, 64)
    return (w1, b1, w2, b2, w3)


def _prep_h_mlp(layers):
    return (_pad2(layers[0]['W'], 128, 128), _pad_row(layers[0]['b'], 128),
            _pad2(layers[1]['W'], 128, 64))


def _prep_g_mlp(layers):
    return (_pad2(layers[0]['W'], 64, 128), _pad_row(layers[0]['b'], 128),
            _pad2(layers[1]['W'], 128, 128))


def kernel(dists_same, dists_anti, dists_ne, senders_same, receivers_same,
           senders_anti, receivers_anti, senders_ne, receivers_ne, params):
    E = dists_same.shape[0]
    ds2 = dists_same.reshape(E, 1)
    da2 = dists_anti.reshape(E, 1)
    dn2 = dists_ne.reshape(E, 1)
    sn2 = senders_ne.reshape(E, 1)
    y128 = _pad2(params['Y'], 128, 64)

    elec = jnp.broadcast_to(params['X'][0], (N_ELEC, EMB))

    for i in range(N_LAYERS):
        lp = params['layers'][i]
        if i == 0:
            we_s = _edge_mlp(ds2, _prep_w_mlp(lp['w_same'], lp['h_same'][0]), E)
            we_a = _edge_mlp(da2, _prep_w_mlp(lp['w_anti'], lp['h_anti'][0]), E)
            weh_s, weh_a = we_s, we_a
        else:
            we_s = _edge_mlp(ds2, _prep_w_mlp(lp['w_same']), E)
            we_a = _edge_mlp(da2, _prep_w_mlp(lp['w_anti']), E)
            h_s, h_a = _node_h(elec, _prep_h_mlp(lp['h_same']) + _prep_h_mlp(lp['h_anti']))
            weh_s = we_s * h_s[senders_same]
            weh_a = we_a * h_a[senders_anti]
        weh_n = _edge_mlp_ne(dn2, sn2, _prep_w_mlp(lp['w_ne']), y128, E)
        z_s = jax.ops.segment_sum(weh_s, receivers_same, num_segments=N_ELEC)
        z_a = jax.ops.segment_sum(weh_a, receivers_anti, num_segments=N_ELEC)
        z_n = jax.ops.segment_sum(weh_n, receivers_ne, num_segments=N_ELEC)
        elec = _update(elec, z_s, z_a, z_n,
                       _prep_g_mlp(lp['g_same']) + _prep_g_mlp(lp['g_anti'])
                       + _prep_g_mlp(lp['g_ne']))
    return elec
